# baseline (speedup 1.0000x reference)
import functools

import jax
import jax.numpy as jnp
from jax import lax
from jax.experimental import pallas as pl
from jax.experimental.pallas import tpu as pltpu
from jax.experimental.pallas import tpu_sc as plsc

jax.config.update("jax_enable_x64", True)

_NC = 2
_NS = 16
_NW = _NC * _NS
_LANES = 16
_ROWS = 11 * 64
_RPW = 24
_PADROWS = _NW * _RPW
_STR = 6
_WIN = _RPW * _STR
_NWORDS = _ROWS * _STR
_TAILB = _NWORDS - _WIN
_TROWS = _ROWS - 29 * _RPW

_mesh = plsc.VectorSubcoreMesh(core_axis_name="c", subcore_axis_name="s")


@functools.partial(
    pl.kernel,
    mesh=_mesh,
    out_type=[
        jax.ShapeDtypeStruct((_PADROWS, 128), jnp.int32),
        jax.ShapeDtypeStruct((11, 128), jnp.float32),
        jax.ShapeDtypeStruct((11,), jnp.int32),
    ],
    scratch_types=[
        pltpu.VMEM((_WIN,), jnp.int32),
        pltpu.VMEM((_RPW, 128), jnp.int32),
        pltpu.VMEM((11, 128), jnp.int32),
        pltpu.VMEM((16,), jnp.int32),
        pltpu.VMEM((16, 128), jnp.float32),
        pltpu.SemaphoreType.DMA,
    ],
)
def _sc_mask_gather(thr_hbm, sel_hbm, table_hbm, mask_out, rows_out, sel_out,
                    thr_v, mask_v, selbuf_v, idx_v, rows_v, sem):
    wid = lax.axis_index("s") * _NC + lax.axis_index("c")
    col0 = lax.iota(jnp.int32, _LANES)

    @pl.when(wid == 0)
    def _gather_start():
        pltpu.sync_copy(sel_hbm.at[pl.ds(0, 11), pl.ds(0, 128)], selbuf_v)
        v = jnp.full((_LANES,), jnp.int32(0))
        for i in range(11):
            e = selbuf_v[i, pl.ds(0, _LANES)][4]
            v = jnp.where(col0 == i, jnp.full((_LANES,), e, jnp.int32), v)
        idx_v[pl.ds(0, _LANES)] = v
        pltpu.make_async_copy(table_hbm.at[idx_v], rows_v, sem).start()

    def mask_rows(nrows, first_pos):
        vecs = [thr_v[pl.ds(b * _LANES, _LANES)]
                for b in range(_WIN // _LANES)]
        for r in range(nrows):
            pos = first_pos + r * _STR
            t = vecs[pos // _LANES][pos % _LANES]
            tvec = jnp.full((_LANES,), t, jnp.int32)
            for k in range(128 // _LANES):
                val = jnp.where(col0 + (k * _LANES) < tvec,
                                jnp.int32(1), jnp.int32(0))
                mask_v[r, pl.ds(k * _LANES, _LANES)] = val

    @pl.when(wid <= 28)
    def _mask_main():
        pltpu.sync_copy(thr_hbm.at[pl.ds(wid * _WIN, _WIN)], thr_v)
        mask_rows(_RPW, 0)
        pltpu.sync_copy(mask_v, mask_out.at[pl.ds(wid * _RPW, _RPW)])

    @pl.when(wid == 29)
    def _mask_tail():
        pltpu.sync_copy(thr_hbm.at[pl.ds(jnp.int32(_TAILB), _WIN)], thr_v)
        mask_rows(_TROWS, 29 * _RPW * _STR - _TAILB)
        pltpu.sync_copy(mask_v.at[pl.ds(0, _TROWS)],
                        mask_out.at[pl.ds(jnp.int32(29 * _RPW), _TROWS)])

    @pl.when(wid == 0)
    def _gather_finish():
        pltpu.make_async_copy(table_hbm.at[idx_v], rows_v, sem).wait()
        pltpu.sync_copy(rows_v.at[pl.ds(0, 11)], rows_out)
        pltpu.sync_copy(idx_v.at[pl.ds(0, 11)], sel_out)


def kernel(primals_1, primals_2, primals_3, primals_4):
    ct1 = primals_3.astype(jnp.int32)
    pairs = lax.bitcast_convert_type(primals_2, jnp.int32)
    thr_flat = pairs.reshape(_NWORDS)
    sel_view = pairs.reshape(11, 384)
    mask_i32, index, select_2 = _sc_mask_gather(thr_flat, sel_view, primals_4)
    lt = mask_i32[:_ROWS, :120].astype(jnp.bool_).reshape(11, 64, 120)
    z0 = jnp.zeros((11, 6, 128), jnp.float64)
    z1 = jnp.zeros((11, 32, 128), jnp.float64)
    z2 = jnp.zeros((11, 128), jnp.float64)
    return (primals_1, ct1, z0, z1, z2, lt, index, select_2)


# speedup vs baseline: 1.0780x; 1.0780x over previous
import functools

import jax
import jax.numpy as jnp
from jax import lax
from jax.experimental import pallas as pl
from jax.experimental.pallas import tpu as pltpu
from jax.experimental.pallas import tpu_sc as plsc

jax.config.update("jax_enable_x64", True)

_NC = 2
_NS = 16
_NW = _NC * _NS
_LANES = 16
_ROWS = 11 * 64
_RPW = 24
_PADROWS = _NW * _RPW

_mesh = plsc.VectorSubcoreMesh(core_axis_name="c", subcore_axis_name="s")


@functools.partial(
    pl.kernel,
    mesh=_mesh,
    out_type=[
        jax.ShapeDtypeStruct((_PADROWS, 128), jnp.int32),
        jax.ShapeDtypeStruct((11, 128), jnp.float32),
        jax.ShapeDtypeStruct((11,), jnp.int32),
    ],
    scratch_types=[
        pltpu.VMEM((_RPW,), jnp.int32),
        pltpu.VMEM((_RPW, 128), jnp.int32),
        pltpu.VMEM((16,), jnp.int32),
        pltpu.VMEM((16, 128), jnp.float32),
        pltpu.SemaphoreType.DMA,
    ],
)
def _sc_mask_gather(prep_hbm, table_hbm, mask_out, rows_out, sel_out,
                    thr_v, mask_v, idx_v, rows_v, sem):
    wid = lax.axis_index("s") * _NC + lax.axis_index("c")
    base = wid * _RPW

    @pl.when(wid == 0)
    def _gather_start():
        pltpu.sync_copy(prep_hbm.at[pl.ds(jnp.int32(_PADROWS), 16)], idx_v)
        pltpu.make_async_copy(table_hbm.at[idx_v], rows_v, sem).start()

    pltpu.sync_copy(prep_hbm.at[pl.ds(base, _RPW)], thr_v)
    col0 = lax.iota(jnp.int32, _LANES)

    blk0 = thr_v[pl.ds(0, _LANES)]
    blk1 = thr_v[pl.ds(_RPW - _LANES, _LANES)]
    for r in range(_RPW):
        t = blk0[r] if r < _LANES else blk1[r - (_RPW - _LANES)]
        tvec = jnp.full((_LANES,), t, jnp.int32)
        for k in range(128 // _LANES):
            col = col0 + (k * _LANES)
            val = jnp.where(col < tvec, jnp.int32(1), jnp.int32(0))
            mask_v[r, pl.ds(k * _LANES, _LANES)] = val

    pltpu.sync_copy(mask_v, mask_out.at[pl.ds(base, _RPW)])

    @pl.when(wid == 0)
    def _gather_finish():
        pltpu.make_async_copy(table_hbm.at[idx_v], rows_v, sem).wait()
        pltpu.sync_copy(rows_v.at[pl.ds(0, 11)], rows_out)
        pltpu.sync_copy(idx_v.at[pl.ds(0, 11)], sel_out)


def kernel(primals_1, primals_2, primals_3, primals_4):
    p2 = primals_2.astype(jnp.int32)
    ct1 = primals_3.astype(jnp.int32)
    thr = jnp.pad(p2[:, :, 0].reshape(-1), (0, _PADROWS - _ROWS))
    idx16 = jnp.pad(p2[:, 0, 2], (0, 16 - 11))
    prep = jnp.concatenate([thr, idx16])
    mask_i32, index, select_2 = _sc_mask_gather(prep, primals_4)
    lt = mask_i32[:_ROWS, :120].astype(jnp.bool_).reshape(11, 64, 120)
    z0 = jnp.zeros((11, 6, 128), jnp.float64)
    z1 = jnp.zeros((11, 32, 128), jnp.float64)
    z2 = jnp.zeros((11, 128), jnp.float64)
    return (primals_1, ct1, z0, z1, z2, lt, index, select_2)


# speedup vs baseline: 1.1287x; 1.0471x over previous
import functools

import jax
import jax.numpy as jnp
from jax import lax
from jax.experimental import pallas as pl
from jax.experimental.pallas import tpu as pltpu
from jax.experimental.pallas import tpu_sc as plsc

jax.config.update("jax_enable_x64", True)

_NC = 1
_NS = 16
_NW = _NC * _NS
_LANES = 16
_ROWS = 11 * 64
_RPW = 48
_PADROWS = _NW * _RPW

_mesh = plsc.VectorSubcoreMesh(core_axis_name="c", subcore_axis_name="s", num_cores=1)


@functools.partial(
    pl.kernel,
    mesh=_mesh,
    out_type=[
        jax.ShapeDtypeStruct((_PADROWS, 128), jnp.int32),
        jax.ShapeDtypeStruct((11, 128), jnp.float32),
        jax.ShapeDtypeStruct((11,), jnp.int32),
    ],
    scratch_types=[
        pltpu.VMEM((_RPW,), jnp.int32),
        pltpu.VMEM((_RPW, 128), jnp.int32),
        pltpu.VMEM((16,), jnp.int32),
        pltpu.VMEM((16, 128), jnp.float32),
        pltpu.SemaphoreType.DMA,
    ],
)
def _sc_mask_gather(prep_hbm, table_hbm, mask_out, rows_out, sel_out,
                    thr_v, mask_v, idx_v, rows_v, sem):
    wid = lax.axis_index("s") * _NC + lax.axis_index("c")
    base = wid * _RPW

    @pl.when(wid == 0)
    def _gather_start():
        pltpu.sync_copy(prep_hbm.at[pl.ds(jnp.int32(_PADROWS), 16)], idx_v)
        pltpu.make_async_copy(table_hbm.at[idx_v], rows_v, sem).start()

    pltpu.sync_copy(prep_hbm.at[pl.ds(base, _RPW)], thr_v)
    col0 = lax.iota(jnp.int32, _LANES)

    blks = [thr_v[pl.ds(b * _LANES, _LANES)] for b in range(_RPW // _LANES)]
    for r in range(_RPW):
        t = blks[r // _LANES][r % _LANES]
        tvec = jnp.full((_LANES,), t, jnp.int32)
        for k in range(128 // _LANES):
            col = col0 + (k * _LANES)
            val = jnp.where(col < tvec, jnp.int32(1), jnp.int32(0))
            mask_v[r, pl.ds(k * _LANES, _LANES)] = val

    pltpu.sync_copy(mask_v, mask_out.at[pl.ds(base, _RPW)])

    @pl.when(wid == 0)
    def _gather_finish():
        pltpu.make_async_copy(table_hbm.at[idx_v], rows_v, sem).wait()
        pltpu.sync_copy(rows_v.at[pl.ds(0, 11)], rows_out)
        pltpu.sync_copy(idx_v.at[pl.ds(0, 11)], sel_out)


def kernel(primals_1, primals_2, primals_3, primals_4):
    p2 = primals_2.astype(jnp.int32)
    ct1 = primals_3.astype(jnp.int32)
    thr = jnp.pad(p2[:, :, 0].reshape(-1), (0, _PADROWS - _ROWS))
    idx16 = jnp.pad(p2[:, 0, 2], (0, 16 - 11))
    prep = jnp.concatenate([thr, idx16])
    mask_i32, index, select_2 = _sc_mask_gather(prep, primals_4)
    lt = mask_i32[:_ROWS, :120].astype(jnp.bool_).reshape(11, 64, 120)
    z0 = jnp.zeros((11, 6, 128), jnp.float64)
    z1 = jnp.zeros((11, 32, 128), jnp.float64)
    z2 = jnp.zeros((11, 128), jnp.float64)
    return (primals_1, ct1, z0, z1, z2, lt, index, select_2)


# speedup vs baseline: 1.1363x; 1.0067x over previous
import functools

import jax
import jax.numpy as jnp
from jax import lax
from jax.experimental import pallas as pl
from jax.experimental.pallas import tpu as pltpu
from jax.experimental.pallas import tpu_sc as plsc

jax.config.update("jax_enable_x64", True)

_NC = 1
_NS = 16
_NW = _NC * _NS
_LANES = 16
_ROWS = 11 * 64
_RPW = 48
_PADROWS = _NW * _RPW

_mesh = plsc.VectorSubcoreMesh(core_axis_name="c", subcore_axis_name="s", num_cores=1)


@functools.partial(
    pl.kernel,
    mesh=_mesh,
    out_type=[
        jax.ShapeDtypeStruct((_PADROWS, 128), jnp.int32),
        jax.ShapeDtypeStruct((11, 128), jnp.float32),
        jax.ShapeDtypeStruct((11,), jnp.int32),
    ],
    scratch_types=[
        pltpu.VMEM((_RPW,), jnp.int32),
        pltpu.VMEM((_RPW, 128), jnp.int32),
        pltpu.VMEM((16,), jnp.int32),
        pltpu.VMEM((16, 128), jnp.float32),
        pltpu.SemaphoreType.DMA,
    ],
)
def _sc_mask_gather(prep_hbm, table_hbm, mask_out, rows_out, sel_out,
                    thr_v, mask_v, idx_v, rows_v, sem):
    wid = lax.axis_index("s") * _NC + lax.axis_index("c")

    @pl.when(wid == 0)
    def _gather_start():
        pltpu.sync_copy(prep_hbm.at[pl.ds(jnp.int32(_PADROWS), 16)], idx_v)
        pltpu.make_async_copy(table_hbm.at[idx_v], rows_v, sem).start()

    col0 = lax.iota(jnp.int32, _LANES)

    @pl.when(wid >= 1)
    def _mask():
        mbase = (wid - 1) * _RPW
        pltpu.sync_copy(prep_hbm.at[pl.ds(mbase, _RPW)], thr_v)
        blks = [thr_v[pl.ds(b * _LANES, _LANES)]
                for b in range(_RPW // _LANES)]
        for r in range(_RPW):
            t = blks[r // _LANES][r % _LANES]
            tvec = jnp.full((_LANES,), t, jnp.int32)
            for k in range(128 // _LANES):
                col = col0 + (k * _LANES)
                val = jnp.where(col < tvec, jnp.int32(1), jnp.int32(0))
                mask_v[r, pl.ds(k * _LANES, _LANES)] = val
        pltpu.sync_copy(mask_v, mask_out.at[pl.ds(mbase, _RPW)])

    @pl.when(wid == 0)
    def _gather_finish():
        pltpu.make_async_copy(table_hbm.at[idx_v], rows_v, sem).wait()
        pltpu.sync_copy(rows_v.at[pl.ds(0, 11)], rows_out)
        pltpu.sync_copy(idx_v.at[pl.ds(0, 11)], sel_out)


def kernel(primals_1, primals_2, primals_3, primals_4):
    p2 = primals_2.astype(jnp.int32)
    ct1 = primals_3.astype(jnp.int32)
    thr = jnp.pad(p2[:, :, 0].reshape(-1), (0, _PADROWS - _ROWS))
    idx16 = jnp.pad(p2[:, 0, 2], (0, 16 - 11))
    prep = jnp.concatenate([thr, idx16])
    mask_i32, index, select_2 = _sc_mask_gather(prep, primals_4)
    lt = mask_i32[:_ROWS, :120].astype(jnp.bool_).reshape(11, 64, 120)
    z0 = jnp.zeros((11, 6, 128), jnp.float64)
    z1 = jnp.zeros((11, 32, 128), jnp.float64)
    z2 = jnp.zeros((11, 128), jnp.float64)
    return (primals_1, ct1, z0, z1, z2, lt, index, select_2)
